# baseline (speedup 1.0000x reference)
import functools
import jax
import jax.numpy as jnp
from jax import lax
from jax.experimental import pallas as pl
from jax.experimental.pallas import tpu as pltpu

_N = 10000
_NP = 10240
_E0 = 170000
_EB = 1024
_NEB = 170
_H = 6
_D = 128
_F = _H * _D
_FC = 256
_NFC = _F // _FC
_HPC = _FC // _D
_G = 64


def _leaky(x):
    return jnp.where(x >= 0, x, 0.2 * x)


def _dense_body(x_ref, w_ref, as_ref, ad_ref, h_ref, aso_ref, ado_ref, eco_ref):
    h = jnp.dot(x_ref[...], w_ref[...], preferred_element_type=jnp.float32)
    h_ref[...] = h
    h3 = h.reshape(h.shape[0], _H, _D)
    asv = jnp.sum(h3 * as_ref[...][None], axis=-1)
    adv = jnp.sum(h3 * ad_ref[...][None], axis=-1)
    z2 = jnp.zeros((h.shape[0], 2), jnp.float32)
    aso_ref[...] = jnp.concatenate([asv, z2], axis=1)
    ado_ref[...] = jnp.concatenate([adv, z2], axis=1)
    eco_ref[...] = jnp.concatenate([_leaky(asv + adv), z2], axis=1)


def _dense(x, W, a_s, a_d):
    k = x.shape[1]
    nb = _NP // 256
    return pl.pallas_call(
        _dense_body,
        grid=(nb,),
        in_specs=[
            pl.BlockSpec((256, k), lambda i: (i, 0)),
            pl.BlockSpec((k, _F), lambda i: (0, 0)),
            pl.BlockSpec((_H, _D), lambda i: (0, 0)),
            pl.BlockSpec((_H, _D), lambda i: (0, 0)),
        ],
        out_specs=[
            pl.BlockSpec((256, _F), lambda i: (i, 0)),
            pl.BlockSpec((256, 8), lambda i: (i, 0)),
            pl.BlockSpec((256, 8), lambda i: (i, 0)),
            pl.BlockSpec((256, 8), lambda i: (i, 0)),
        ],
        out_shape=[
            jax.ShapeDtypeStruct((_NP, _F), jnp.float32),
            jax.ShapeDtypeStruct((_NP, 8), jnp.float32),
            jax.ShapeDtypeStruct((_NP, 8), jnp.float32),
            jax.ShapeDtypeStruct((_NP, 8), jnp.float32),
        ],
    )(x, W, a_s, a_d)


def _edge_body0(src_ref, dst_ref, h_ref, as_ref, ad_ref, ec_ref, b_ref,
                out_ref, den_ref):
    eb = pl.program_id(0)

    @pl.when(eb == 0)
    def _():
        out_ref[...] = jnp.zeros_like(out_ref)
        den_ref[...] = jnp.zeros_like(den_ref)

    def body(i, carry):
        s = src_ref[0, 0, i]
        d = dst_ref[0, 0, i]
        e = _leaky(as_ref[pl.ds(s, 1), :] + ad_ref[pl.ds(d, 1), :])
        valid = ((eb * _EB + i) < _E0).astype(jnp.float32)
        w = jnp.exp(e - ec_ref[pl.ds(d, 1), :]) * valid
        den_ref[pl.ds(d, 1), :] += w
        wv = jnp.concatenate(
            [jnp.broadcast_to(w[:, 0:1], (1, _D)),
             jnp.broadcast_to(w[:, 1:2], (1, _D))], axis=1)
        out_ref[pl.ds(d, 1), :] += h_ref[pl.ds(s, 1), :] * wv
        return carry

    lax.fori_loop(0, _EB, body, 0)

    @pl.when(eb == _NEB - 1)
    def _():
        den = den_ref[...]
        dsel = jnp.concatenate(
            [jnp.broadcast_to(den[:, 0:1], (_NP, _D)),
             jnp.broadcast_to(den[:, 1:2], (_NP, _D))], axis=1) + 1e-16
        out_ref[...] = jnp.maximum(out_ref[...] / dsel + b_ref[...], 0.0)


def _make_edge_bodyk(fc):
    c0, c1 = _HPC * fc, _HPC * fc + 1

    def body_k(src_ref, dst_ref, h_ref, as_ref, ad_ref, ec_ref, b_ref,
               den_ref, out_ref):
        eb = pl.program_id(0)

        @pl.when(eb == 0)
        def _():
            out_ref[...] = jnp.zeros_like(out_ref)

        def body(i, carry):
            s = src_ref[0, 0, i]
            d = dst_ref[0, 0, i]
            e = _leaky(as_ref[pl.ds(s, 1), :] + ad_ref[pl.ds(d, 1), :])
            valid = ((eb * _EB + i) < _E0).astype(jnp.float32)
            w = jnp.exp(e - ec_ref[pl.ds(d, 1), :]) * valid
            wv = jnp.concatenate(
                [jnp.broadcast_to(w[:, c0:c0 + 1], (1, _D)),
                 jnp.broadcast_to(w[:, c1:c1 + 1], (1, _D))], axis=1)
            out_ref[pl.ds(d, 1), :] += h_ref[pl.ds(s, 1), :] * wv
            return carry

        lax.fori_loop(0, _EB, body, 0)

        @pl.when(eb == _NEB - 1)
        def _():
            den = den_ref[...]
            dsel = jnp.concatenate(
                [jnp.broadcast_to(den[:, c0:c0 + 1], (_NP, _D)),
                 jnp.broadcast_to(den[:, c1:c1 + 1], (_NP, _D))],
                axis=1) + 1e-16
            out_ref[...] = jnp.maximum(out_ref[...] / dsel + b_ref[...], 0.0)

    return body_k


def _edge(src3, dst3, h, as_, ad_, ec_, b):
    idx_spec = pl.BlockSpec((1, 1, _EB), lambda eb: (eb, 0, 0),
                            memory_space=pltpu.SMEM)
    small_spec = pl.BlockSpec((_NP, 8), lambda eb: (0, 0))

    def hchunk_spec(fc):
        return pl.BlockSpec((_NP, _FC), lambda eb, fc=fc: (0, fc))

    def bchunk_spec(fc):
        return pl.BlockSpec((1, _FC), lambda eb, fc=fc: (0, fc))

    out0, den = pl.pallas_call(
        _edge_body0,
        grid=(_NEB,),
        in_specs=[idx_spec, idx_spec, hchunk_spec(0), small_spec, small_spec,
                  small_spec, bchunk_spec(0)],
        out_specs=[pl.BlockSpec((_NP, _FC), lambda eb: (0, 0)),
                   pl.BlockSpec((_NP, 8), lambda eb: (0, 0))],
        out_shape=[jax.ShapeDtypeStruct((_NP, _FC), jnp.float32),
                   jax.ShapeDtypeStruct((_NP, 8), jnp.float32)],
    )(src3, dst3, h, as_, ad_, ec_, b)

    chunks = [out0]
    for fc in range(1, _NFC):
        outk = pl.pallas_call(
            _make_edge_bodyk(fc),
            grid=(_NEB,),
            in_specs=[idx_spec, idx_spec, hchunk_spec(fc), small_spec,
                      small_spec, small_spec, bchunk_spec(fc), small_spec],
            out_specs=pl.BlockSpec((_NP, _FC), lambda eb: (0, 0)),
            out_shape=jax.ShapeDtypeStruct((_NP, _FC), jnp.float32),
        )(src3, dst3, h, as_, ad_, ec_, b, den)
        chunks.append(outk)
    return jnp.concatenate(chunks, axis=1), den


def _conv(x, src3, dst3, W, a_s, a_d, b):
    h, as_, ad_, ec_ = _dense(x, W, a_s, a_d)
    out, _ = _edge(src3, dst3, h, as_, ad_, ec_, b.reshape(1, _F))
    return out


def _pool_body(batch_ref, hc_ref, hs_ref, wr1_ref, br1_ref, wr2_ref, br2_ref,
               pooled_ref, xg_ref, cnt_ref):
    i = pl.program_id(0)

    @pl.when(i == 0)
    def _():
        pooled_ref[...] = jnp.zeros_like(pooled_ref)
        cnt_ref[...] = jnp.zeros_like(cnt_ref)

    h = hc_ref[...] + hs_ref[...]
    bids = batch_ref[...].reshape(1, _EB)
    oh = (bids == lax.broadcasted_iota(jnp.int32, (_G, _EB), 0)
          ).astype(jnp.float32)
    pooled_ref[...] += jnp.dot(oh, h, preferred_element_type=jnp.float32)
    cnt_ref[...] += jnp.broadcast_to(
        jnp.sum(oh, axis=1, keepdims=True), (_G, _D))

    @pl.when(i == _NP // _EB - 1)
    def _():
        cl = jnp.maximum(cnt_ref[...][:, 0:1], 1.0)
        pooled = pooled_ref[...] / cl
        pooled_ref[...] = pooled
        y = jnp.maximum(
            jnp.dot(pooled, wr1_ref[...], preferred_element_type=jnp.float32)
            + br1_ref[...], 0.0)
        xg_ref[...] = jnp.dot(
            y, wr2_ref[...], preferred_element_type=jnp.float32) + br2_ref[...]


def _pool(batch3, hc, hs, Wr1, br1, Wr2, br2):
    nb = _NP // _EB
    return pl.pallas_call(
        _pool_body,
        grid=(nb,),
        in_specs=[
            pl.BlockSpec((1, 1, _EB), lambda i: (i, 0, 0)),
            pl.BlockSpec((_EB, _F), lambda i: (i, 0)),
            pl.BlockSpec((_EB, _F), lambda i: (i, 0)),
            pl.BlockSpec((_F, _D), lambda i: (0, 0)),
            pl.BlockSpec((1, _D), lambda i: (0, 0)),
            pl.BlockSpec((_D, _D), lambda i: (0, 0)),
            pl.BlockSpec((1, _D), lambda i: (0, 0)),
        ],
        out_specs=[
            pl.BlockSpec((_G, _F), lambda i: (0, 0)),
            pl.BlockSpec((_G, _D), lambda i: (0, 0)),
        ],
        out_shape=[
            jax.ShapeDtypeStruct((_G, _F), jnp.float32),
            jax.ShapeDtypeStruct((_G, _D), jnp.float32),
        ],
        scratch_shapes=[pltpu.VMEM((_G, _D), jnp.float32)],
    )(batch3, hc, hs, Wr1, br1.reshape(1, _D), Wr2, br2.reshape(1, _D))


def kernel(x, edge_index, batch, W_c1, asrc_c1, adst_c1, b_c1, W_c2, asrc_c2,
           adst_c2, b_c2, W_s1, asrc_s1, adst_s1, b_s1, W_s2, asrc_s2,
           adst_s2, b_s2, Wr1, br1, Wr2, br2):
    n = x.shape[0]
    loop = jnp.arange(n, dtype=edge_index.dtype)
    src = jnp.concatenate([edge_index[0], loop]).astype(jnp.int32)
    dst = jnp.concatenate([edge_index[1], loop]).astype(jnp.int32)
    epad = _NEB * _EB - src.shape[0]
    src3 = jnp.pad(src, (0, epad)).reshape(_NEB, 1, _EB)
    dst3 = jnp.pad(dst, (0, epad)).reshape(_NEB, 1, _EB)

    xp = jnp.pad(x, ((0, _NP - n), (0, 0)))
    batch3 = jnp.pad(batch.astype(jnp.int32), (0, _NP - n),
                     constant_values=_G + 1).reshape(_NP // _EB, 1, _EB)

    xs = _conv(xp, src3, dst3, W_s1, asrc_s1, adst_s1, b_s1)
    xs = _conv(xs, src3, dst3, W_s2, asrc_s2, adst_s2, b_s2)
    h = _conv(xp, src3, dst3, W_c1, asrc_c1, adst_c1, b_c1)
    h = _conv(h, src3, dst3, W_c2, asrc_c2, adst_c2, b_c2)

    pooled, xg = _pool(batch3, h, xs, Wr1, br1, Wr2, br2)
    return (jnp.squeeze(xg), pooled)
